# Initial kernel scaffold; baseline (speedup 1.0000x reference)
import jax
import jax.numpy as jnp
from jax.experimental import pallas as pl


def kernel(input, weights):
    raise NotImplementedError("write your pallas kernel here")


# speedup vs baseline: 9.1788x; 9.1788x over previous
import functools
import math

import jax
import jax.numpy as jnp
from jax.experimental import pallas as pl


def _body(w_ref, o_ref, *, shift, scale, half):
    w = w_ref[...]
    ws = w[:, :half]
    wc = w[:, half:]
    j = jax.lax.broadcasted_iota(jnp.int32, (1, half), 1).astype(jnp.float32)
    ang = shift * jnp.exp(j * (-scale))
    c = jnp.cos(ang)
    s = jnp.sin(ang)
    o_ref[0] = w
    o_ref[1] = jnp.concatenate([ws * c + wc * s, wc * c - ws * s], axis=1)


def kernel(input, weights):
    n, dim = weights.shape
    half = dim // 2
    seq_len = input.shape[1]
    scale = math.log(10000.0) / (half - 1)
    rows_per_step = 256
    out = pl.pallas_call(
        functools.partial(_body, shift=float(n), scale=scale, half=half),
        grid=(n // rows_per_step,),
        in_specs=[pl.BlockSpec((rows_per_step, dim), lambda i: (i, 0))],
        out_specs=pl.BlockSpec((2, rows_per_step, dim), lambda i: (0, i, 0)),
        out_shape=jax.ShapeDtypeStruct((2, n, dim), jnp.float32),
    )(weights)
    return jax.lax.stop_gradient(out.reshape(seq_len, dim))


# speedup vs baseline: 10.7442x; 1.1705x over previous
import functools
import math

import jax
import jax.numpy as jnp
from jax.experimental import pallas as pl


def _body(w_ref, o_ref, *, rows, scale, half):
    shift = (pl.program_id(0) * rows).astype(jnp.float32)
    w = w_ref[...]
    ws = w[:, :half]
    wc = w[:, half:]
    j = jax.lax.broadcasted_iota(jnp.int32, (1, half), 1).astype(jnp.float32)
    ang = shift * jnp.exp(j * (-scale))
    c = jnp.cos(ang)
    s = jnp.sin(ang)
    o_ref[...] = jnp.concatenate([ws * c + wc * s, wc * c - ws * s], axis=1)


def kernel(input, weights):
    _, dim = weights.shape
    half = dim // 2
    seq_len = input.shape[1]
    scale = math.log(10000.0) / (half - 1)
    rows = 256
    out = pl.pallas_call(
        functools.partial(_body, rows=rows, scale=scale, half=half),
        grid=(seq_len // rows,),
        in_specs=[pl.BlockSpec((rows, dim), lambda i: (0, 0))],
        out_specs=pl.BlockSpec((rows, dim), lambda i: (i, 0)),
        out_shape=jax.ShapeDtypeStruct((seq_len, dim), jnp.float32),
    )(weights)
    return jax.lax.stop_gradient(out)


# speedup vs baseline: 10.7985x; 1.0050x over previous
import functools
import math

import jax
import jax.numpy as jnp
from jax.experimental import pallas as pl


def _body(w_ref, o_ref, *, rows, scale, half):
    shift = (pl.program_id(0) * rows).astype(jnp.float32)
    w = w_ref[...]
    ws = w[:, :half]
    wc = w[:, half:]
    j = jax.lax.broadcasted_iota(jnp.int32, (1, half), 1).astype(jnp.float32)
    ang = shift * jnp.exp(j * (-scale))
    c = jnp.cos(ang)
    s = jnp.sin(ang)
    o_ref[:, :half] = ws * c + wc * s
    o_ref[:, half:] = wc * c - ws * s


def kernel(input, weights):
    _, dim = weights.shape
    half = dim // 2
    seq_len = input.shape[1]
    scale = math.log(10000.0) / (half - 1)
    rows = 256
    out = pl.pallas_call(
        functools.partial(_body, rows=rows, scale=scale, half=half),
        grid=(seq_len // rows,),
        in_specs=[pl.BlockSpec((rows, dim), lambda i: (0, 0))],
        out_specs=pl.BlockSpec((rows, dim), lambda i: (i, 0)),
        out_shape=jax.ShapeDtypeStruct((seq_len, dim), jnp.float32),
    )(weights)
    return jax.lax.stop_gradient(out)
